# baseline (speedup 1.0000x reference)
import functools

import jax
import jax.numpy as jnp
from jax import lax
from jax.experimental import pallas as pl
from jax.experimental.pallas import tpu as pltpu
from jax.experimental.pallas import tpu_sc as plsc

D = 512
FAST = 16
SLOW = 64
B = 32
T = 128
STEPS = T - 3
_HI = lax.Precision.HIGHEST


def _scan_body(h_ref, wgd_ref, bgd_ref, fidx_ref, sidx_ref, fused_ref,
               sused_ref, sc_ref):
    bgd = bgd_ref[...]
    wgd_bf = wgd_ref[...].astype(jnp.bfloat16)
    v = h_ref[...].reshape(B * T, D).astype(jnp.bfloat16)
    s = jnp.dot(v, wgd_bf, preferred_element_type=jnp.float32) + bgd
    ws = jax.nn.sigmoid(s[:, 0:1])
    comb = jnp.concatenate([ws, s[:, 1:2]], 1)
    sc_ref[...] = jnp.swapaxes(comb.reshape(B, T, 2), 0, 1).reshape(T * B, 2)

    iota16 = lax.broadcasted_iota(jnp.int32, (B, FAST), 1)
    iota64 = lax.broadcasted_iota(jnp.int32, (B, SLOW), 1)

    def step(t, carry):
        fast_score, fast_tok, slow_tok, nfast, ndem = carry
        s = sc_ref[pl.ds(t * B, B), :]
        w = s[:, 0:1] >= 0.4
        d = s[:, 1:2]
        full = nfast >= FAST
        m = jnp.min(fast_score, axis=1, keepdims=True)
        jmin = jnp.min(jnp.where(fast_score == m, iota16, FAST), axis=1,
                       keepdims=True)
        slot = jnp.where(full, jmin, nfast)
        onehot_f = (iota16 == slot) & w
        victim = jnp.sum(jnp.where(iota16 == jmin, fast_tok, 0), axis=1,
                         keepdims=True)
        demote = w & full
        ring = jnp.bitwise_and(ndem, SLOW - 1)
        onehot_s = (iota64 == ring) & demote
        slow_tok = jnp.where(onehot_s, victim, slow_tok)
        fast_score = jnp.where(onehot_f, d, fast_score)
        fast_tok = jnp.where(onehot_f, t, fast_tok)
        nfast = nfast + (w & ~full).astype(jnp.int32)
        ndem = ndem + demote.astype(jnp.int32)
        return fast_score, fast_tok, slow_tok, nfast, ndem

    init = (jnp.zeros((B, FAST), jnp.float32), jnp.zeros((B, FAST), jnp.int32),
            jnp.zeros((B, SLOW), jnp.int32), jnp.zeros((B, 1), jnp.int32),
            jnp.zeros((B, 1), jnp.int32))
    _, fast_tok, slow_tok, nfast, ndem = lax.fori_loop(0, STEPS, step, init)

    fused = iota16 < nfast
    sused = iota64 < jnp.minimum(ndem, SLOW)
    fidx_ref[...] = jnp.where(fused, fast_tok, 0)
    sidx_ref[...] = jnp.where(sused, slow_tok, 0)
    fused_ref[...] = fused.astype(jnp.float32)
    sused_ref[...] = sused.astype(jnp.float32)


def _gather_body(h_ref, fidx_ref, sidx_ref, fused_ref, sused_ref,
                 fout_ref, sout_ref):
    b = pl.program_id(0)
    hb = h_ref[0]
    iota_tf = lax.broadcasted_iota(jnp.int32, (T, FAST), 0)
    iota_ts = lax.broadcasted_iota(jnp.int32, (T, SLOW), 0)
    ftok = fidx_ref[pl.ds(b, 1), :]
    stok = sidx_ref[pl.ds(b, 1), :]
    fmask = fused_ref[pl.ds(b, 1), :]
    smask = sused_ref[pl.ds(b, 1), :]
    pf = jnp.where(iota_tf == ftok, 1.0, 0.0) * fmask
    ps = jnp.where(iota_ts == stok, 1.0, 0.0) * smask
    dn = (((0,), (0,)), ((), ()))
    fout_ref[0] = lax.dot_general(pf, hb, dn, precision=_HI)
    sout_ref[0] = lax.dot_general(ps, hb, dn, precision=_HI)


_SLOTS = FAST + SLOW


def _sc_gather_body(h2_hbm, fidx_hbm, sidx_hbm, fmsk_hbm, smsk_hbm,
                    fout_hbm, sout_hbm, idx_v, msk_v, rows_v, sem):
    wid = lax.axis_index("s") * 2 + lax.axis_index("c")
    pltpu.sync_copy(fidx_hbm.at[pl.ds(wid * FAST, FAST)],
                    idx_v.at[pl.ds(0, FAST)])
    pltpu.sync_copy(sidx_hbm.at[pl.ds(wid * SLOW, SLOW)],
                    idx_v.at[pl.ds(FAST, SLOW)])
    pltpu.sync_copy(fmsk_hbm.at[pl.ds(wid * FAST, FAST)],
                    msk_v.at[pl.ds(0, FAST)])
    pltpu.sync_copy(smsk_hbm.at[pl.ds(wid * SLOW, SLOW)],
                    msk_v.at[pl.ds(FAST, SLOW)])
    base = wid * T
    for j in range(_SLOTS // 16):
        idx_v[pl.ds(j * 16, 16)] = idx_v[pl.ds(j * 16, 16)] + base
    pltpu.async_copy(h2_hbm.at[idx_v], rows_v, sem).wait()

    def row_fn(r, _):
        m = msk_v[pl.ds(r, 16)][0]
        for c in range(D // 16):
            rows_v[r, pl.ds(c * 16, 16)] = rows_v[r, pl.ds(c * 16, 16)] * m
        return 0

    lax.fori_loop(0, _SLOTS, row_fn, 0)

    pltpu.sync_copy(rows_v.at[pl.ds(0, FAST)], fout_hbm.at[wid])
    pltpu.sync_copy(rows_v.at[pl.ds(FAST, SLOW)], sout_hbm.at[wid])


@functools.partial(
    pl.kernel,
    mesh=plsc.VectorSubcoreMesh(core_axis_name="c", subcore_axis_name="s"),
    out_type=[
        jax.ShapeDtypeStruct((B, FAST, D), jnp.float32),
        jax.ShapeDtypeStruct((B, SLOW, D), jnp.float32),
    ],
    scratch_types=[
        pltpu.VMEM((_SLOTS,), jnp.int32),
        pltpu.VMEM((_SLOTS + 16,), jnp.float32),
        pltpu.VMEM((_SLOTS, D), jnp.float32),
        pltpu.SemaphoreType.DMA,
    ],
)
def _sc_gather(h2_hbm, fidx_hbm, sidx_hbm, fmsk_hbm, smsk_hbm,
               fout_hbm, sout_hbm, idx_v, msk_v, rows_v, sem):
    _sc_gather_body(h2_hbm, fidx_hbm, sidx_hbm, fmsk_hbm, smsk_hbm,
                    fout_hbm, sout_hbm, idx_v, msk_v, rows_v, sem)


@jax.jit
def kernel(h, wg, bg, wd, bd):
    wgd = jnp.stack([wg, wd], axis=1)
    bgd = jnp.stack([jnp.asarray(bg, jnp.float32),
                     jnp.asarray(bd, jnp.float32)]).reshape(1, 2)

    fidx, sidx, fused, sused = pl.pallas_call(
        _scan_body,
        out_shape=[
            jax.ShapeDtypeStruct((B, FAST), jnp.int32),
            jax.ShapeDtypeStruct((B, SLOW), jnp.int32),
            jax.ShapeDtypeStruct((B, FAST), jnp.float32),
            jax.ShapeDtypeStruct((B, SLOW), jnp.float32),
        ],
        scratch_shapes=[pltpu.VMEM((T * B, 2), jnp.float32)],
    )(h, wgd, bgd)

    fast_mem, slow_mem = _sc_gather(
        h.reshape(B * T, D), fidx.reshape(B * FAST), sidx.reshape(B * SLOW),
        fused.reshape(B * FAST), sused.reshape(B * SLOW))

    return fast_mem, slow_mem, fused, sused


# speedup vs baseline: 2.0288x; 2.0288x over previous
import functools

import jax
import jax.numpy as jnp
from jax import lax
from jax.experimental import pallas as pl
from jax.experimental.pallas import tpu as pltpu
from jax.experimental.pallas import tpu_sc as plsc

D = 512
FAST = 16
SLOW = 64
B = 32
T = 128
STEPS = T - 3
_SLOTS = FAST + SLOW


def _score_body(h_ref, wgd_ref, bgd_ref, sc_ref):
    bgd = bgd_ref[...]
    wgd_bf = wgd_ref[...].astype(jnp.bfloat16)
    v = h_ref[...].reshape(B * T, D).astype(jnp.bfloat16)
    s = jnp.dot(v, wgd_bf, preferred_element_type=jnp.float32) + bgd
    ws = jax.nn.sigmoid(s[:, 0:1])
    sc_ref[...] = jnp.concatenate([ws, s[:, 1:2]], 1)


def _sc_body(h2_hbm, sc_hbm, fout_hbm, sout_hbm, fused_hbm, sused_hbm,
             sco_v, idx_v, um_v, rows_v, sem):
    wid = lax.axis_index("s") * 2 + lax.axis_index("c")
    pltpu.sync_copy(sc_hbm.at[pl.ds(wid * 2 * T, 2 * T)],
                    sco_v.at[pl.ds(0, 2 * T)])

    iota16 = lax.broadcasted_iota(jnp.int32, (16,), 0)
    zeros16 = jnp.zeros((16,), jnp.int32)

    for j in range(_SLOTS // 16):
        idx_v[pl.ds(j * 16, 16)] = zeros16

    gdn = lax.GatherDimensionNumbers(offset_dims=(), collapsed_slice_dims=(0,),
                                     start_index_map=(0,))

    def _perm(x, idx):
        return lax.gather(x, idx[:, None], gdn, (1,),
                          mode=lax.GatherScatterMode.PROMISE_IN_BOUNDS)

    def _bfly_min(x):
        for off in (8, 4, 2, 1):
            x = jnp.minimum(x, _perm(x, jnp.bitwise_xor(iota16, off)))
        return x

    iota16f = iota16.astype(jnp.float32)
    zerosf = jnp.zeros((16,), jnp.float32)

    def step(t, carry):
        fs, ft, nfast, ndem = carry
        wv = sco_v[pl.ds(2 * t, 16)]
        gate01 = (wv[0] >= 0.4).astype(jnp.int32)
        full01 = (nfast >= FAST).astype(jnp.int32)
        mnv = _bfly_min(fs)
        jmf = _bfly_min(jnp.where(fs == mnv, iota16f, 16.0))
        jmv = jmf.astype(jnp.int32)
        victim = _perm(ft, jmv)
        f_iv = zeros16 + full01
        slot_i = f_iv * jmv + (1 - f_iv) * (zeros16 + nfast)
        slot_eff = slot_i * gate01 + FAST * (1 - gate01)
        dv = zerosf + wv[1]
        fs = jnp.where(iota16f == slot_eff.astype(jnp.float32), dv, fs)
        ft = jnp.where(iota16 == slot_eff, zeros16 + t, ft)
        demote01 = gate01 * full01
        ring = jnp.bitwise_and(ndem, SLOW - 1)
        ro = jnp.bitwise_and(ring, 15)
        rbase = FAST + ring - ro
        ro_eff = zeros16 + (ro * demote01 + 16 * (1 - demote01))
        old = idx_v[pl.ds(rbase, 16)]
        idx_v[pl.ds(rbase, 16)] = jnp.where(iota16 == ro_eff, victim, old)
        nfast = nfast + gate01 * (1 - full01)
        ndem = ndem + demote01
        return fs, ft, nfast, ndem

    init = (jnp.zeros((16,), jnp.float32), jnp.zeros((16,), jnp.int32),
            jnp.int32(0), jnp.int32(0))
    _, ft, nfast, ndem = lax.fori_loop(0, STEPS, step, init)

    nsl = jnp.minimum(ndem, SLOW)
    nfast_f = nfast.astype(jnp.float32)
    nsl_f = nsl.astype(jnp.float32)
    base = wid * T
    idx_v[pl.ds(0, 16)] = jnp.where(iota16 < (zeros16 + nfast), ft, 0) + base
    um_v[pl.ds(0, 16)] = jnp.where(iota16f < (zerosf + nfast_f), 1.0, 0.0)
    for j in range(SLOW // 16):
        sl = idx_v[pl.ds(FAST + j * 16, 16)]
        smask_i = (iota16 + j * 16) < (zeros16 + nsl)
        idx_v[pl.ds(FAST + j * 16, 16)] = jnp.where(smask_i, sl, 0) + base
        um_v[pl.ds(FAST + j * 16, 16)] = jnp.where(
            (iota16f + float(j * 16)) < (zerosf + nsl_f), 1.0, 0.0)

    pltpu.async_copy(h2_hbm.at[idx_v], rows_v, sem).wait()

    zrow = jnp.zeros((16,), jnp.float32)

    def row_fn(r, _):
        @pl.when(um_v[pl.ds(r, 16)][0] == 0.0)
        def _():
            for c in range(D // 16):
                rows_v[r, pl.ds(c * 16, 16)] = zrow
        return 0

    lax.fori_loop(0, _SLOTS, row_fn, 0)

    pltpu.sync_copy(rows_v.at[pl.ds(0, FAST)], fout_hbm.at[wid])
    pltpu.sync_copy(rows_v.at[pl.ds(FAST, SLOW)], sout_hbm.at[wid])
    pltpu.sync_copy(um_v.at[pl.ds(0, FAST)],
                    fused_hbm.at[pl.ds(wid * FAST, FAST)])
    pltpu.sync_copy(um_v.at[pl.ds(FAST, SLOW)],
                    sused_hbm.at[pl.ds(wid * SLOW, SLOW)])


@functools.partial(
    pl.kernel,
    mesh=plsc.VectorSubcoreMesh(core_axis_name="c", subcore_axis_name="s"),
    out_type=[
        jax.ShapeDtypeStruct((B, FAST, D), jnp.float32),
        jax.ShapeDtypeStruct((B, SLOW, D), jnp.float32),
        jax.ShapeDtypeStruct((B * FAST,), jnp.float32),
        jax.ShapeDtypeStruct((B * SLOW,), jnp.float32),
    ],
    scratch_types=[
        pltpu.VMEM((2 * T + 16,), jnp.float32),
        pltpu.VMEM((_SLOTS,), jnp.int32),
        pltpu.VMEM((_SLOTS + 16,), jnp.float32),
        pltpu.VMEM((_SLOTS, D), jnp.float32),
        pltpu.SemaphoreType.DMA,
    ],
)
def _sc_scan_gather(h2_hbm, sc_hbm, fout_hbm, sout_hbm, fused_hbm, sused_hbm,
                    sco_v, idx_v, um_v, rows_v, sem):
    _sc_body(h2_hbm, sc_hbm, fout_hbm, sout_hbm, fused_hbm, sused_hbm,
             sco_v, idx_v, um_v, rows_v, sem)


@jax.jit
def kernel(h, wg, bg, wd, bd):
    wgd = jnp.stack([wg, wd], axis=1)
    bgd = jnp.stack([jnp.asarray(bg, jnp.float32),
                     jnp.asarray(bd, jnp.float32)]).reshape(1, 2)

    scores = pl.pallas_call(
        _score_body,
        out_shape=jax.ShapeDtypeStruct((B * T, 2), jnp.float32),
    )(h, wgd, bgd)

    fast_mem, slow_mem, fused, sused = _sc_scan_gather(
        h.reshape(B * T, D), scores.reshape(B * T * 2))

    return (fast_mem, slow_mem,
            fused.reshape(B, FAST), sused.reshape(B, SLOW))
